# baseline (speedup 1.0000x reference)
import functools

import jax
import jax.numpy as jnp
from jax import lax
from jax.experimental import pallas as pl
from jax.experimental.pallas import tpu as pltpu
from jax.experimental.pallas import tpu_sc as plsc

NC, NS = 2, 16
NW = NC * NS
SEQ = 200
DIM = 64
LANES = 16
SEQS_PER_CHUNK = 2
CHUNK = SEQS_PER_CHUNK * SEQ


def _sc_embed(idx_flat, wordp_i32, pos_table):
    n_rows = idx_flat.shape[0]
    rows_per_w = n_rows // NW
    n_chunks = rows_per_w // CHUNK
    assert n_chunks % 2 == 0
    mesh = plsc.VectorSubcoreMesh(core_axis_name="c", subcore_axis_name="s")

    @functools.partial(
        pl.kernel,
        out_type=jax.ShapeDtypeStruct((n_rows, DIM), jnp.float32),
        mesh=mesh,
        scratch_types=[
            pltpu.VMEM((CHUNK,), jnp.int32),
            pltpu.VMEM((CHUNK,), jnp.int32),
            pltpu.VMEM((CHUNK, DIM // 2), jnp.int32),
            pltpu.VMEM((CHUNK, DIM // 2), jnp.int32),
            pltpu.VMEM((CHUNK, DIM), jnp.float32),
            pltpu.VMEM((CHUNK, DIM), jnp.float32),
            pltpu.VMEM((SEQ, DIM), jnp.float32),
            pltpu.SemaphoreType.DMA,
            pltpu.SemaphoreType.DMA,
            pltpu.SemaphoreType.DMA,
            pltpu.SemaphoreType.DMA,
        ],
        compiler_params=pltpu.CompilerParams(
            use_tc_tiling_on_sc=False, needs_layout_passes=False),
    )
    def k(idx_hbm, word_hbm, pos_hbm, out_hbm,
          i0, i1, w0, w1, b0, b1, pos_v, g0, g1, s0, s1):
        idx_vs = (i0, i1)
        wbufs = (w0, w1)
        bufs = (b0, b1)
        gsems = (g0, g1)
        ssems = (s0, s1)
        wid = lax.axis_index("s") * NC + lax.axis_index("c")
        wbase = wid * rows_per_w
        pltpu.sync_copy(pos_hbm, pos_v)

        def gather_start(c, b):
            base = wbase + c * CHUNK
            pltpu.sync_copy(idx_hbm.at[pl.ds(base, CHUNK)], idx_vs[b])
            pltpu.async_copy(word_hbm.at[idx_vs[b]], wbufs[b], gsems[b])

        def gather_wait(b):
            pltpu.make_async_copy(
                word_hbm.at[idx_vs[b]], wbufs[b], gsems[b]).wait()

        def scatter_start(c, b):
            base = wbase + c * CHUNK
            pltpu.async_copy(bufs[b], out_hbm.at[pl.ds(base, CHUNK)], ssems[b])

        def scatter_wait(c, b):
            base = wbase + c * CHUNK
            pltpu.make_async_copy(
                bufs[b], out_hbm.at[pl.ds(base, CHUNK)], ssems[b]).wait()

        def widen_add_pos(b):
            wbuf = wbufs[b]
            buf = bufs[b]
            himask = jnp.full((LANES,), -65536, jnp.int32)

            def row_body(pr, rcarry):
                for s in range(SEQS_PER_CHUNK):
                    r = s * SEQ + pr
                    for h in range(2):
                        w = wbuf[r, pl.ds(h * LANES, LANES)]
                        lo = plsc.bitcast(w << 16, jnp.float32)
                        hi = plsc.bitcast(w & himask, jnp.float32)
                        ca = pl.ds(32 * h, LANES)
                        cb = pl.ds(32 * h + LANES, LANES)
                        buf[r, ca] = lo + pos_v[pr, ca]
                        buf[r, cb] = hi + pos_v[pr, cb]
                return rcarry

            lax.fori_loop(0, SEQ, row_body, 0)

        gather_start(0, 0)

        def pair_body(p, carry):
            for b in range(2):
                c = p * 2 + b
                nb = 1 - b

                @pl.when(c + 1 < n_chunks)
                def _():
                    @pl.when(c >= 1)
                    def _():
                        scatter_wait(c - 1, nb)

                    gather_start(c + 1, nb)

                gather_wait(b)
                widen_add_pos(b)
                scatter_start(c, b)
            return carry

        lax.fori_loop(0, n_chunks // 2, pair_body, 0)
        scatter_wait(n_chunks - 2, 0)
        scatter_wait(n_chunks - 1, 1)

    return k(idx_flat, wordp_i32, pos_table)


def kernel(inputs, word_table, pos_table):
    batch, seq = inputs.shape
    idx_flat = inputs.reshape(batch * seq).astype(jnp.int32)
    vocab = word_table.shape[0]
    wp = word_table.reshape(vocab, 2, 2, LANES)
    wp = wp.transpose(0, 1, 3, 2).reshape(vocab, DIM)
    wordp = wp.astype(jnp.bfloat16)
    wordp_i32 = jax.lax.bitcast_convert_type(
        wordp.reshape(vocab, DIM // 2, 2), jnp.int32)
    out = _sc_embed(idx_flat, wordp_i32, pos_table)
    return out.reshape(batch, seq, DIM)


# speedup vs baseline: 1.2946x; 1.2946x over previous
import functools

import jax
import jax.numpy as jnp
from jax import lax
from jax.experimental import pallas as pl
from jax.experimental.pallas import tpu as pltpu
from jax.experimental.pallas import tpu_sc as plsc

NC, NS = 2, 16
NW = NC * NS
SEQ = 200
DIM = 64
LANES = 16
SEQS_PER_CHUNK = 4
CHUNK = SEQS_PER_CHUNK * SEQ


def _sc_embed(idx_flat, word_table, pos_table):
    n_rows = idx_flat.shape[0]
    rows_per_w = n_rows // NW
    n_chunks = rows_per_w // CHUNK
    assert n_chunks % 2 == 0
    mesh = plsc.VectorSubcoreMesh(core_axis_name="c", subcore_axis_name="s")

    @functools.partial(
        pl.kernel,
        out_type=jax.ShapeDtypeStruct((n_rows, DIM), jnp.float32),
        mesh=mesh,
        scratch_types=[
            pltpu.VMEM((CHUNK,), jnp.int32),
            pltpu.VMEM((CHUNK,), jnp.int32),
            pltpu.VMEM((CHUNK, DIM), jnp.float32),
            pltpu.VMEM((CHUNK, DIM), jnp.float32),
            pltpu.VMEM((SEQ, DIM), jnp.float32),
            pltpu.SemaphoreType.DMA,
            pltpu.SemaphoreType.DMA,
            pltpu.SemaphoreType.DMA,
            pltpu.SemaphoreType.DMA,
        ],
        compiler_params=pltpu.CompilerParams(use_tc_tiling_on_sc=False),
    )
    def k(idx_hbm, word_hbm, pos_hbm, out_hbm,
          i0, i1, b0, b1, pos_v, g0, g1, s0, s1):
        idx_vs = (i0, i1)
        bufs = (b0, b1)
        gsems = (g0, g1)
        ssems = (s0, s1)
        wid = lax.axis_index("s") * NC + lax.axis_index("c")
        wbase = wid * rows_per_w
        pltpu.sync_copy(pos_hbm, pos_v)

        def gather_start(c, b):
            base = wbase + c * CHUNK
            pltpu.sync_copy(idx_hbm.at[pl.ds(base, CHUNK)], idx_vs[b])
            pltpu.async_copy(word_hbm.at[idx_vs[b]], bufs[b], gsems[b])

        def gather_wait(b):
            pltpu.make_async_copy(
                word_hbm.at[idx_vs[b]], bufs[b], gsems[b]).wait()

        def scatter_start(c, b):
            base = wbase + c * CHUNK
            pltpu.async_copy(bufs[b], out_hbm.at[pl.ds(base, CHUNK)], ssems[b])

        def scatter_wait(c, b):
            base = wbase + c * CHUNK
            pltpu.make_async_copy(
                bufs[b], out_hbm.at[pl.ds(base, CHUNK)], ssems[b]).wait()

        def add_pos(b):
            buf = bufs[b]

            def row_body(pr, rcarry):
                for s in range(SEQS_PER_CHUNK):
                    r = s * SEQ + pr
                    for j in range(DIM // LANES):
                        col = pl.ds(j * LANES, LANES)
                        buf[r, col] = buf[r, col] + pos_v[pr, col]
                return rcarry

            lax.fori_loop(0, SEQ, row_body, 0)

        gather_start(0, 0)

        def pair_body(p, carry):
            for b in range(2):
                c = p * 2 + b
                nb = 1 - b

                @pl.when(c + 1 < n_chunks)
                def _():
                    @pl.when(c >= 1)
                    def _():
                        scatter_wait(c - 1, nb)

                    gather_start(c + 1, nb)

                gather_wait(b)
                add_pos(b)
                scatter_start(c, b)
            return carry

        lax.fori_loop(0, n_chunks // 2, pair_body, 0)
        scatter_wait(n_chunks - 2, 0)
        scatter_wait(n_chunks - 1, 1)

    return k(idx_flat, word_table, pos_table)


def kernel(inputs, word_table, pos_table):
    batch, seq = inputs.shape
    idx_flat = inputs.reshape(batch * seq).astype(jnp.int32)
    out = _sc_embed(idx_flat, word_table, pos_table)
    return out.reshape(batch, seq, DIM)
